# Initial kernel scaffold; baseline (speedup 1.0000x reference)
import jax
import jax.numpy as jnp
from jax.experimental import pallas as pl


def kernel(x, e, hyperedge_index, ln_scale, ln_bias, W, b):
    raise NotImplementedError("write your pallas kernel here")


# speedup vs baseline: 8.9123x; 8.9123x over previous
import functools

import jax
import jax.numpy as jnp
from jax import lax
from jax.experimental import pallas as pl
from jax.experimental.pallas import tpu as pltpu
from jax.experimental.pallas import tpu_sc as plsc

_D = 2
_F = 128
_DSQ = _D * _D
_TBLW = 16
_NC = 2
_NS = 16
_NW = _NC * _NS
_CH = 128
_L = 16


def _tc_tables_body(x_ref, e_ref, wx_ref, we_ref, tn_ref, te_ref):
    lane = lax.broadcasted_iota(jnp.int32, tn_ref.shape, 1)
    for src, w, dst in ((x_ref, wx_ref, tn_ref), (e_ref, we_ref, te_ref)):
        v = src[...]
        m = (v[:, :_F] + v[:, _F:]) * 0.5
        p = jnp.dot(m, w[...], preferred_element_type=jnp.float32)
        q = jnp.sum(m * m, axis=1, keepdims=True)
        dst[...] = p + jnp.where(lane == 5, q, 0.0)


def _tc_tables(xr, er, wx, we):
    n, nh = xr.shape[0], er.shape[0]
    blk = 1000
    return pl.pallas_call(
        _tc_tables_body,
        grid=(n // blk,),
        in_specs=[
            pl.BlockSpec((blk, _D * _F), lambda i: (i, 0)),
            pl.BlockSpec((blk, _D * _F), lambda i: (i, 0)),
            pl.BlockSpec((_F, _TBLW), lambda i: (0, 0)),
            pl.BlockSpec((_F, _TBLW), lambda i: (0, 0)),
        ],
        out_specs=[
            pl.BlockSpec((blk, _TBLW), lambda i: (i, 0)),
            pl.BlockSpec((blk, _TBLW), lambda i: (i, 0)),
        ],
        out_shape=[
            jax.ShapeDtypeStruct((n, _TBLW), jnp.float32),
            jax.ShapeDtypeStruct((nh, _TBLW), jnp.float32),
        ],
    )(xr, er, wx, we)


def _sc_sheaf(tn, te, rowp, colp, consts, nnzp):
    per_tile = nnzp // _NW
    nchunk = per_tile // _CH
    outlen = nnzp * _DSQ
    olen_t = per_tile * _DSQ
    mesh = plsc.VectorSubcoreMesh(core_axis_name="c", subcore_axis_name="s")

    @functools.partial(
        pl.kernel,
        mesh=mesh,
        compiler_params=pltpu.CompilerParams(needs_layout_passes=False,
                                             use_tc_tiling_on_sc=False),
        out_type=[
            jax.ShapeDtypeStruct((outlen,), jnp.int32),
            jax.ShapeDtypeStruct((outlen,), jnp.int32),
            jax.ShapeDtypeStruct((outlen,), jnp.float32),
        ],
        scratch_types=[
            pltpu.VMEM((nchunk, _CH), jnp.int32),
            pltpu.VMEM((nchunk, _CH), jnp.int32),
            pltpu.VMEM((_CH, _TBLW), jnp.float32),
            pltpu.VMEM((_CH, _TBLW), jnp.float32),
            pltpu.VMEM((8 * _L,), jnp.float32),
            pltpu.VMEM((olen_t,), jnp.int32),
            pltpu.VMEM((olen_t,), jnp.int32),
            pltpu.VMEM((olen_t,), jnp.float32),
            pltpu.SemaphoreType.DMA,
            pltpu.SemaphoreType.DMA,
        ],
    )
    def sck(tn_hbm, te_hbm, row_hbm, col_hbm, cst_hbm,
            i0_hbm, i1_hbm, at_hbm,
            rbuf, cbuf, ngb, egb, cst, i0b, i1b, atb, sem_n, sem_e):
        wid = lax.axis_index("s") * _NC + lax.axis_index("c")
        pltpu.sync_copy(row_hbm.at[pl.ds(wid * nchunk, nchunk)], rbuf)
        pltpu.sync_copy(col_hbm.at[pl.ds(wid * nchunk, nchunk)], cbuf)
        pltpu.sync_copy(cst_hbm, cst)
        lanes = lax.iota(jnp.int32, _L)

        def chunk(j, carry):
            cpn = pltpu.async_copy(tn_hbm.at[rbuf.at[j]], ngb, sem_n)
            cpe = pltpu.async_copy(te_hbm.at[cbuf.at[j]], egb, sem_e)
            cpn.wait()
            cpe.wait()
            base = j * (_CH * _DSQ)
            for g in range(_CH // _L):
                rows = lanes + (g * _L)
                r = rbuf[j, pl.ds(g * _L, _L)]
                c = cbuf[j, pl.ds(g * _L, _L)]

                def col(refv, k):
                    return plsc.load_gather(
                        refv, [rows, jnp.full((_L,), k, jnp.int32)])

                sx = col(ngb, 4)
                qx = col(ngb, 5)
                se = col(egb, 4)
                qe = col(egb, 5)
                mu = (sx + se) * (1.0 / (2.0 * _F))
                var = (qx + qe) * (1.0 / (2.0 * _F)) - mu * mu
                v = jnp.maximum(var, 0.0) + 1e-5
                iv = plsc.bitcast(v, jnp.int32)
                seed = jnp.full((_L,), 0x5F3759DF, jnp.int32)
                y = plsc.bitcast(seed - lax.shift_right_logical(iv, 1),
                                 jnp.float32)
                for _ in range(3):
                    y = y * (1.5 - 0.5 * v * y * y)
                pos = base + (g * _L * _DSQ) + lanes * _DSQ
                r2 = r * 2
                c2 = c * 2
                for jj in range(_DSQ):
                    pxj = col(ngb, jj)
                    pej = col(egb, jj)
                    tj = cst[pl.ds(jj * _L, _L)]
                    cj = cst[pl.ds((_DSQ + jj) * _L, _L)]
                    z = (pxj + pej - mu * tj) * y + cj
                    sg = 1.0 / (1.0 + jnp.exp(-z))
                    idxv = pos + jj
                    plsc.store_scatter(atb, [idxv], sg)
                    plsc.store_scatter(i0b, [idxv], r2 + (jj >> 1))
                    plsc.store_scatter(i1b, [idxv], c2 + (jj & 1))
            return carry

        lax.fori_loop(0, nchunk, chunk, 0)
        obase = wid * olen_t
        pltpu.sync_copy(i0b, i0_hbm.at[pl.ds(obase, olen_t)])
        pltpu.sync_copy(i1b, i1_hbm.at[pl.ds(obase, olen_t)])
        pltpu.sync_copy(atb, at_hbm.at[pl.ds(obase, olen_t)])

    return sck(tn, te, rowp, colp, consts)


def kernel(x, e, hyperedge_index, ln_scale, ln_bias, W, b):
    n_nodes = x.shape[0] // _D
    n_he = e.shape[0] // _D
    nnz = hyperedge_index.shape[1]
    Ws = W * ln_scale[:, None]
    t = jnp.sum(Ws, axis=0)
    cb = ln_bias @ W + b
    wx = (jnp.zeros((_F, _TBLW), jnp.float32)
          .at[:, :_DSQ].set(Ws[:_F]).at[:, 4].set(1.0))
    we = (jnp.zeros((_F, _TBLW), jnp.float32)
          .at[:, :_DSQ].set(Ws[_F:]).at[:, 4].set(1.0))
    xr = x.reshape(n_nodes, _D * _F)
    er = e.reshape(n_he, _D * _F)
    tn, te = _tc_tables(xr, er, wx, we)
    consts = jnp.repeat(jnp.concatenate([t, cb]), _L)

    grain = _NW * _CH
    nnzp = ((nnz + grain - 1) // grain) * grain
    pad = nnzp - nnz
    row = hyperedge_index[0]
    col = hyperedge_index[1]
    rowp = jnp.concatenate([row, jnp.zeros((pad,), row.dtype)])
    colp = jnp.concatenate([col, jnp.zeros((pad,), col.dtype)])
    rowp = rowp.reshape(nnzp // _CH, _CH).astype(jnp.int32)
    colp = colp.reshape(nnzp // _CH, _CH).astype(jnp.int32)

    i0, i1, at = _sc_sheaf(tn, te, rowp, colp, consts, nnzp)
    m = nnz * _DSQ
    idx = jnp.stack([i0[:m], i1[:m]])
    return idx, at[:m]


# speedup vs baseline: 9.9523x; 1.1167x over previous
import functools

import jax
import jax.numpy as jnp
from jax import lax
from jax.experimental import pallas as pl
from jax.experimental.pallas import tpu as pltpu
from jax.experimental.pallas import tpu_sc as plsc

_D = 2
_F = 128
_DSQ = _D * _D
_TBLW = 16
_NC = 2
_NS = 16
_NW = _NC * _NS
_CH = 128
_L = 16
_NBUF = 4


def _tc_tables_body(x_ref, e_ref, wx_ref, we_ref, tn_ref, te_ref):
    lane = lax.broadcasted_iota(jnp.int32, tn_ref.shape, 1)
    for src, w, dst in ((x_ref, wx_ref, tn_ref), (e_ref, we_ref, te_ref)):
        v = src[...]
        m = (v[:, :_F] + v[:, _F:]) * 0.5
        p = jnp.dot(m, w[...], preferred_element_type=jnp.float32)
        q = jnp.sum(m * m, axis=1, keepdims=True)
        dst[...] = p + jnp.where(lane == 5, q, 0.0)


def _tc_tables(xr, er, wx, we):
    n, nh = xr.shape[0], er.shape[0]
    blk = 1000
    return pl.pallas_call(
        _tc_tables_body,
        grid=(n // blk,),
        in_specs=[
            pl.BlockSpec((blk, _D * _F), lambda i: (i, 0)),
            pl.BlockSpec((blk, _D * _F), lambda i: (i, 0)),
            pl.BlockSpec((_F, _TBLW), lambda i: (0, 0)),
            pl.BlockSpec((_F, _TBLW), lambda i: (0, 0)),
        ],
        out_specs=[
            pl.BlockSpec((blk, _TBLW), lambda i: (i, 0)),
            pl.BlockSpec((blk, _TBLW), lambda i: (i, 0)),
        ],
        out_shape=[
            jax.ShapeDtypeStruct((n, _TBLW), jnp.float32),
            jax.ShapeDtypeStruct((nh, _TBLW), jnp.float32),
        ],
    )(xr, er, wx, we)


def _sc_sheaf(tn, te, rowp, colp, consts, nnzp):
    per_tile = nnzp // _NW
    nchunk = per_tile // _CH
    outlen = nnzp * _DSQ
    olen_t = per_tile * _DSQ
    mesh = plsc.VectorSubcoreMesh(core_axis_name="c", subcore_axis_name="s")

    @functools.partial(
        pl.kernel,
        mesh=mesh,
        compiler_params=pltpu.CompilerParams(needs_layout_passes=False,
                                             use_tc_tiling_on_sc=False),
        out_type=[
            jax.ShapeDtypeStruct((outlen,), jnp.int32),
            jax.ShapeDtypeStruct((outlen,), jnp.int32),
            jax.ShapeDtypeStruct((outlen,), jnp.float32),
        ],
        scratch_types=[
            pltpu.VMEM((nchunk, _CH), jnp.int32),
            pltpu.VMEM((nchunk, _CH), jnp.int32),
            pltpu.VMEM((_NBUF, _CH, _TBLW), jnp.float32),
            pltpu.VMEM((_NBUF, _CH, _TBLW), jnp.float32),
            pltpu.VMEM((8 * _L,), jnp.float32),
            pltpu.VMEM((olen_t,), jnp.int32),
            pltpu.VMEM((olen_t,), jnp.int32),
            pltpu.VMEM((olen_t,), jnp.float32),
        ] + [pltpu.SemaphoreType.DMA] * (2 * _NBUF),
    )
    def sck(tn_hbm, te_hbm, row_hbm, col_hbm, cst_hbm,
            i0_hbm, i1_hbm, at_hbm,
            rbuf, cbuf, ngb, egb, cst, i0b, i1b, atb, *sems):
        sem_n = sems[:_NBUF]
        sem_e = sems[_NBUF:]
        wid = lax.axis_index("s") * _NC + lax.axis_index("c")
        pltpu.sync_copy(row_hbm.at[pl.ds(wid * nchunk, nchunk)], rbuf)
        pltpu.sync_copy(col_hbm.at[pl.ds(wid * nchunk, nchunk)], cbuf)
        pltpu.sync_copy(cst_hbm, cst)
        lanes = lax.iota(jnp.int32, _L)

        def fire(j, b):
            pltpu.async_copy(tn_hbm.at[rbuf.at[j]], ngb.at[b], sem_n[b])
            pltpu.async_copy(te_hbm.at[cbuf.at[j]], egb.at[b], sem_e[b])

        def drain(b):
            pltpu.make_async_copy(tn_hbm.at[rbuf.at[0]], ngb.at[b],
                                  sem_n[b]).wait()
            pltpu.make_async_copy(te_hbm.at[cbuf.at[0]], egb.at[b],
                                  sem_e[b]).wait()

        for b in range(_NBUF):
            fire(b, b)

        def compute(j, b):
            base = j * (_CH * _DSQ)
            nb = ngb.at[b]
            eb = egb.at[b]
            for g in range(_CH // _L):
                rows = lanes + (g * _L)
                r = rbuf[j, pl.ds(g * _L, _L)]
                c = cbuf[j, pl.ds(g * _L, _L)]

                def col(refv, k):
                    return plsc.load_gather(
                        refv, [rows, jnp.full((_L,), k, jnp.int32)])

                sx = col(nb, 4)
                qx = col(nb, 5)
                se = col(eb, 4)
                qe = col(eb, 5)
                mu = (sx + se) * (1.0 / (2.0 * _F))
                var = (qx + qe) * (1.0 / (2.0 * _F)) - mu * mu
                v = jnp.maximum(var, 0.0) + 1e-5
                iv = plsc.bitcast(v, jnp.int32)
                seed = jnp.full((_L,), 0x5F3759DF, jnp.int32)
                y = plsc.bitcast(seed - lax.shift_right_logical(iv, 1),
                                 jnp.float32)
                vh = 0.5 * v
                for _ in range(2):
                    y = y * (1.5 - vh * y * y)
                pos = base + (g * _L * _DSQ) + lanes * _DSQ
                r2 = r * 2
                c2 = c * 2
                r3 = r2 + 1
                c3 = c2 + 1
                ivals = ((r2, c2), (r2, c3), (r3, c2), (r3, c3))
                for jj in range(_DSQ):
                    pxj = col(nb, jj)
                    pej = col(eb, jj)
                    tj = cst[pl.ds(jj * _L, _L)]
                    cj = cst[pl.ds((_DSQ + jj) * _L, _L)]
                    z = (pxj + pej - mu * tj) * y + cj
                    sg = 1.0 / (1.0 + jnp.exp(-z))
                    idxv = pos + jj
                    plsc.store_scatter(atb, [idxv], sg)
                    plsc.store_scatter(i0b, [idxv], ivals[jj][0])
                    plsc.store_scatter(i1b, [idxv], ivals[jj][1])

        def round_body(jj, carry):
            j0 = jj * _NBUF
            for b in range(_NBUF):
                j = j0 + b
                drain(b)
                compute(j, b)
                jn = j + _NBUF

                @pl.when(jn < nchunk)
                def _():
                    fire(jn, b)
            return carry

        lax.fori_loop(0, nchunk // _NBUF, round_body, 0)
        obase = wid * olen_t
        pltpu.sync_copy(i0b, i0_hbm.at[pl.ds(obase, olen_t)])
        pltpu.sync_copy(i1b, i1_hbm.at[pl.ds(obase, olen_t)])
        pltpu.sync_copy(atb, at_hbm.at[pl.ds(obase, olen_t)])

    return sck(tn, te, rowp, colp, consts)


def kernel(x, e, hyperedge_index, ln_scale, ln_bias, W, b):
    n_nodes = x.shape[0] // _D
    n_he = e.shape[0] // _D
    nnz = hyperedge_index.shape[1]
    Ws = W * ln_scale[:, None]
    t = jnp.sum(Ws, axis=0)
    cb = ln_bias @ W + b
    wx = (jnp.zeros((_F, _TBLW), jnp.float32)
          .at[:, :_DSQ].set(Ws[:_F]).at[:, 4].set(1.0))
    we = (jnp.zeros((_F, _TBLW), jnp.float32)
          .at[:, :_DSQ].set(Ws[_F:]).at[:, 4].set(1.0))
    xr = x.reshape(n_nodes, _D * _F)
    er = e.reshape(n_he, _D * _F)
    tn, te = _tc_tables(xr, er, wx, we)
    consts = jnp.repeat(jnp.concatenate([t, cb]), _L)

    grain = _NW * _CH
    nnzp = ((nnz + grain - 1) // grain) * grain
    pad = nnzp - nnz
    row = hyperedge_index[0]
    col = hyperedge_index[1]
    rowp = jnp.concatenate([row, jnp.zeros((pad,), row.dtype)])
    colp = jnp.concatenate([col, jnp.zeros((pad,), col.dtype)])
    rowp = rowp.reshape(nnzp // _CH, _CH).astype(jnp.int32)
    colp = colp.reshape(nnzp // _CH, _CH).astype(jnp.int32)

    i0, i1, at = _sc_sheaf(tn, te, rowp, colp, consts, nnzp)
    m = nnz * _DSQ
    idx = jnp.stack([i0[:m], i1[:m]])
    return idx, at[:m]


# speedup vs baseline: 11.2478x; 1.1302x over previous
import functools

import jax
import jax.numpy as jnp
from jax import lax
from jax.experimental import pallas as pl
from jax.experimental.pallas import tpu as pltpu
from jax.experimental.pallas import tpu_sc as plsc

_D = 2
_F = 128
_DSQ = _D * _D
_TBLW = 16
_NC = 2
_NS = 16
_NW = _NC * _NS
_CH = 128
_L = 16
_NBUF = 4


def _tc_tables_body(x_ref, e_ref, wx_ref, we_ref, tn_ref, te_ref):
    lane = lax.broadcasted_iota(jnp.int32, tn_ref.shape, 1)
    for src, w, dst in ((x_ref, wx_ref, tn_ref), (e_ref, we_ref, te_ref)):
        v = src[...]
        m = (v[:, :_F] + v[:, _F:]) * 0.5
        p = jnp.dot(m, w[...], preferred_element_type=jnp.float32)
        q = jnp.sum(m * m, axis=1, keepdims=True)
        dst[...] = p + jnp.where(lane == 5, q, 0.0)


def _tc_tables(xr, er, wx, we):
    n, nh = xr.shape[0], er.shape[0]
    blk = 1000
    return pl.pallas_call(
        _tc_tables_body,
        grid=(n // blk,),
        in_specs=[
            pl.BlockSpec((blk, _D * _F), lambda i: (i, 0)),
            pl.BlockSpec((blk, _D * _F), lambda i: (i, 0)),
            pl.BlockSpec((_F, _TBLW), lambda i: (0, 0)),
            pl.BlockSpec((_F, _TBLW), lambda i: (0, 0)),
        ],
        out_specs=[
            pl.BlockSpec((blk, _TBLW), lambda i: (i, 0)),
            pl.BlockSpec((blk, _TBLW), lambda i: (i, 0)),
        ],
        out_shape=[
            jax.ShapeDtypeStruct((n, _TBLW), jnp.float32),
            jax.ShapeDtypeStruct((nh, _TBLW), jnp.float32),
        ],
    )(xr, er, wx, we)


def _tc_idx_body(r_ref, c_ref, e_ref, o0_ref, o1_ref):
    lane = lax.broadcasted_iota(jnp.int32, o0_ref.shape, 1)
    k = lane & 3
    p0 = (k >> 1).astype(jnp.float32)
    p1 = (k & 1).astype(jnp.float32)
    ex = e_ref[...]
    m0 = jnp.dot(r_ref[...], ex, preferred_element_type=jnp.float32,
                 precision=lax.Precision.HIGHEST)
    m1 = jnp.dot(c_ref[...], ex, preferred_element_type=jnp.float32,
                 precision=lax.Precision.HIGHEST)
    o0_ref[...] = (2.0 * m0 + p0).astype(jnp.int32)
    o1_ref[...] = (2.0 * m1 + p1).astype(jnp.int32)


def _tc_idx(rf, cf, e4):
    n = rf.shape[0]
    return pl.pallas_call(
        _tc_idx_body,
        grid=(1,),
        in_specs=[
            pl.BlockSpec((n, _F), lambda i: (0, 0)),
            pl.BlockSpec((n, _F), lambda i: (0, 0)),
            pl.BlockSpec((_F, _DSQ * _F), lambda i: (0, 0)),
        ],
        out_specs=[
            pl.BlockSpec((n, _DSQ * _F), lambda i: (0, 0)),
            pl.BlockSpec((n, _DSQ * _F), lambda i: (0, 0)),
        ],
        out_shape=[
            jax.ShapeDtypeStruct((n, _DSQ * _F), jnp.int32),
            jax.ShapeDtypeStruct((n, _DSQ * _F), jnp.int32),
        ],
    )(rf, cf, e4)


def _sc_sheaf(tn, te, rowp, colp, consts, nnzp):
    per_tile = nnzp // _NW
    nchunk = per_tile // _CH
    outlen = nnzp * _DSQ
    olen_t = per_tile * _DSQ
    mesh = plsc.VectorSubcoreMesh(core_axis_name="c", subcore_axis_name="s")

    @functools.partial(
        pl.kernel,
        mesh=mesh,
        compiler_params=pltpu.CompilerParams(needs_layout_passes=False,
                                             use_tc_tiling_on_sc=False),
        out_type=jax.ShapeDtypeStruct((outlen,), jnp.float32),
        scratch_types=[
            pltpu.VMEM((nchunk, _CH), jnp.int32),
            pltpu.VMEM((nchunk, _CH), jnp.int32),
            pltpu.VMEM((_NBUF, _CH, _TBLW), jnp.float32),
            pltpu.VMEM((_NBUF, _CH, _TBLW), jnp.float32),
            pltpu.VMEM((8 * _L,), jnp.float32),
            pltpu.VMEM((olen_t,), jnp.float32),
        ] + [pltpu.SemaphoreType.DMA] * (2 * _NBUF),
    )
    def sck(tn_hbm, te_hbm, row_hbm, col_hbm, cst_hbm,
            at_hbm,
            rbuf, cbuf, ngb, egb, cst, atb, *sems):
        sem_n = sems[:_NBUF]
        sem_e = sems[_NBUF:]
        wid = lax.axis_index("s") * _NC + lax.axis_index("c")
        pltpu.sync_copy(row_hbm.at[pl.ds(wid * nchunk, nchunk)], rbuf)
        pltpu.sync_copy(col_hbm.at[pl.ds(wid * nchunk, nchunk)], cbuf)
        pltpu.sync_copy(cst_hbm, cst)
        lanes = lax.iota(jnp.int32, _L)
        tc_vecs = [(cst[pl.ds(jj * _L, _L)],
                    cst[pl.ds((_DSQ + jj) * _L, _L)]) for jj in range(_DSQ)]

        def fire(j, b):
            pltpu.async_copy(tn_hbm.at[rbuf.at[j]], ngb.at[b], sem_n[b])
            pltpu.async_copy(te_hbm.at[cbuf.at[j]], egb.at[b], sem_e[b])

        def drain(b):
            pltpu.make_async_copy(tn_hbm.at[rbuf.at[0]], ngb.at[b],
                                  sem_n[b]).wait()
            pltpu.make_async_copy(te_hbm.at[cbuf.at[0]], egb.at[b],
                                  sem_e[b]).wait()

        for b in range(_NBUF):
            fire(b, b)

        def compute(j, b):
            base = j * (_CH * _DSQ)
            nb = ngb.at[b]
            eb = egb.at[b]
            for g in range(_CH // _L):
                rows = lanes + (g * _L)

                def col(refv, k):
                    return plsc.load_gather(
                        refv, [rows, jnp.full((_L,), k, jnp.int32)])

                sx = col(nb, 4)
                qx = col(nb, 5)
                se = col(eb, 4)
                qe = col(eb, 5)
                mu = (sx + se) * (1.0 / (2.0 * _F))
                var = (qx + qe) * (1.0 / (2.0 * _F)) - mu * mu
                v = jnp.maximum(var, 0.0) + 1e-5
                iv = plsc.bitcast(v, jnp.int32)
                seed = jnp.full((_L,), 0x5F3759DF, jnp.int32)
                y = plsc.bitcast(seed - lax.shift_right_logical(iv, 1),
                                 jnp.float32)
                vh = 0.5 * v
                for _ in range(2):
                    y = y * (1.5 - vh * y * y)
                pos = base + (g * _L * _DSQ) + lanes * _DSQ
                for jj in range(_DSQ):
                    pxj = col(nb, jj)
                    pej = col(eb, jj)
                    tj, cj = tc_vecs[jj]
                    z = (pxj + pej - mu * tj) * y + cj
                    sg = 1.0 / (1.0 + jnp.exp(-z))
                    plsc.store_scatter(atb, [pos + jj], sg)

        def round_body(jj, carry):
            j0 = jj * _NBUF
            for b in range(_NBUF):
                j = j0 + b
                drain(b)
                compute(j, b)
                jn = j + _NBUF

                @pl.when(jn < nchunk)
                def _():
                    fire(jn, b)
            return carry

        lax.fori_loop(0, nchunk // _NBUF, round_body, 0)
        obase = wid * olen_t
        pltpu.sync_copy(atb, at_hbm.at[pl.ds(obase, olen_t)])

    return sck(tn, te, rowp, colp, consts)


def kernel(x, e, hyperedge_index, ln_scale, ln_bias, W, b):
    n_nodes = x.shape[0] // _D
    n_he = e.shape[0] // _D
    nnz = hyperedge_index.shape[1]
    Ws = W * ln_scale[:, None]
    t = jnp.sum(Ws, axis=0)
    cb = ln_bias @ W + b
    wx = (jnp.zeros((_F, _TBLW), jnp.float32)
          .at[:, :_DSQ].set(Ws[:_F]).at[:, 4].set(1.0))
    we = (jnp.zeros((_F, _TBLW), jnp.float32)
          .at[:, :_DSQ].set(Ws[_F:]).at[:, 4].set(1.0))
    xr = x.reshape(n_nodes, _D * _F)
    er = e.reshape(n_he, _D * _F)
    tn, te = _tc_tables(xr, er, wx, we)
    consts = jnp.repeat(jnp.concatenate([t, cb]), _L)

    grain = _NW * _CH
    nnzp = ((nnz + grain - 1) // grain) * grain
    pad = nnzp - nnz
    row = hyperedge_index[0]
    col = hyperedge_index[1]
    rowp = jnp.concatenate([row, jnp.zeros((pad,), row.dtype)])
    colp = jnp.concatenate([col, jnp.zeros((pad,), col.dtype)])
    rowp = rowp.reshape(nnzp // _CH, _CH).astype(jnp.int32)
    colp = colp.reshape(nnzp // _CH, _CH).astype(jnp.int32)

    at = _sc_sheaf(tn, te, rowp, colp, consts, nnzp)

    rf = row.astype(jnp.float32).reshape(nnz // _F, _F)
    cf = col.astype(jnp.float32).reshape(nnz // _F, _F)
    e4 = (jnp.arange(_DSQ * _F)[None, :] // _DSQ
          == jnp.arange(_F)[:, None]).astype(jnp.float32)
    i0, i1 = _tc_idx(rf, cf, e4)

    m = nnz * _DSQ
    idx = jnp.stack([i0.reshape(-1), i1.reshape(-1)])
    return idx, at[:m]


# speedup vs baseline: 11.7915x; 1.0483x over previous
import functools

import jax
import jax.numpy as jnp
from jax import lax
from jax.experimental import pallas as pl
from jax.experimental.pallas import tpu as pltpu
from jax.experimental.pallas import tpu_sc as plsc

_D = 2
_F = 128
_DSQ = _D * _D
_TBLW = 16
_NC = 2
_NS = 16
_NW = _NC * _NS
_CH = 128
_L = 16
_NBUF = 4


def _tc_tables_body(x_ref, e_ref, wx_ref, we_ref, tn_ref, te_ref):
    lane = lax.broadcasted_iota(jnp.int32, tn_ref.shape, 1)
    for src, w, dst in ((x_ref, wx_ref, tn_ref), (e_ref, we_ref, te_ref)):
        v = src[...]
        m = (v[:, :_F] + v[:, _F:]) * 0.5
        p = jnp.dot(m, w[...], preferred_element_type=jnp.float32)
        q = jnp.sum(m * m, axis=1, keepdims=True)
        dst[...] = p + jnp.where(lane == 5, q, 0.0)


def _tc_tables(xr, er, wx, we):
    n, nh = xr.shape[0], er.shape[0]
    blk = 1000
    return pl.pallas_call(
        _tc_tables_body,
        grid=(n // blk,),
        in_specs=[
            pl.BlockSpec((blk, _D * _F), lambda i: (i, 0)),
            pl.BlockSpec((blk, _D * _F), lambda i: (i, 0)),
            pl.BlockSpec((_F, _TBLW), lambda i: (0, 0)),
            pl.BlockSpec((_F, _TBLW), lambda i: (0, 0)),
        ],
        out_specs=[
            pl.BlockSpec((blk, _TBLW), lambda i: (i, 0)),
            pl.BlockSpec((blk, _TBLW), lambda i: (i, 0)),
        ],
        out_shape=[
            jax.ShapeDtypeStruct((n, _TBLW), jnp.float32),
            jax.ShapeDtypeStruct((nh, _TBLW), jnp.float32),
        ],
    )(xr, er, wx, we)


def _tc_idx_body(r_ref, c_ref, e_ref, o0_ref, o1_ref):
    lane = lax.broadcasted_iota(jnp.int32, o0_ref.shape, 1)
    k = lane & 3
    p0 = (k >> 1).astype(jnp.float32)
    p1 = (k & 1).astype(jnp.float32)
    ex = e_ref[...]
    m0 = jnp.dot(r_ref[...], ex, preferred_element_type=jnp.float32,
                 precision=lax.Precision.HIGHEST)
    m1 = jnp.dot(c_ref[...], ex, preferred_element_type=jnp.float32,
                 precision=lax.Precision.HIGHEST)
    o0_ref[...] = (2.0 * m0 + p0).astype(jnp.int32)
    o1_ref[...] = (2.0 * m1 + p1).astype(jnp.int32)


def _tc_idx(rf, cf, e4):
    n = rf.shape[0]
    return pl.pallas_call(
        _tc_idx_body,
        grid=(1,),
        in_specs=[
            pl.BlockSpec((n, _F), lambda i: (0, 0)),
            pl.BlockSpec((n, _F), lambda i: (0, 0)),
            pl.BlockSpec((_F, _DSQ * _F), lambda i: (0, 0)),
        ],
        out_specs=[
            pl.BlockSpec((n, _DSQ * _F), lambda i: (0, 0)),
            pl.BlockSpec((n, _DSQ * _F), lambda i: (0, 0)),
        ],
        out_shape=[
            jax.ShapeDtypeStruct((n, _DSQ * _F), jnp.int32),
            jax.ShapeDtypeStruct((n, _DSQ * _F), jnp.int32),
        ],
    )(rf, cf, e4)


def _sc_sheaf(tn, te, rowp, colp, consts, nnzp):
    per_tile = nnzp // _NW
    nchunk = per_tile // _CH
    outlen = nnzp * _DSQ
    olen_t = per_tile * _DSQ
    mesh = plsc.VectorSubcoreMesh(core_axis_name="c", subcore_axis_name="s")

    @functools.partial(
        pl.kernel,
        mesh=mesh,
        compiler_params=pltpu.CompilerParams(needs_layout_passes=False,
                                             use_tc_tiling_on_sc=False),
        out_type=jax.ShapeDtypeStruct((outlen,), jnp.float32),
        scratch_types=[
            pltpu.VMEM((nchunk, _CH), jnp.int32),
            pltpu.VMEM((nchunk, _CH), jnp.int32),
            pltpu.VMEM((_NBUF, _CH, _TBLW), jnp.float32),
            pltpu.VMEM((_NBUF, _CH, _TBLW), jnp.float32),
            pltpu.VMEM((8 * _L,), jnp.float32),
            pltpu.VMEM((olen_t,), jnp.float32),
        ] + [pltpu.SemaphoreType.DMA] * (2 * _NBUF),
    )
    def sck(tn_hbm, te_hbm, row_hbm, col_hbm, cst_hbm,
            at_hbm,
            rbuf, cbuf, ngb, egb, cst, atb, *sems):
        sem_n = sems[:_NBUF]
        sem_e = sems[_NBUF:]
        wid = lax.axis_index("s") * _NC + lax.axis_index("c")
        pltpu.sync_copy(row_hbm.at[pl.ds(wid * nchunk, nchunk)], rbuf)
        pltpu.sync_copy(col_hbm.at[pl.ds(wid * nchunk, nchunk)], cbuf)
        pltpu.sync_copy(cst_hbm, cst)
        lanes = lax.iota(jnp.int32, _L)
        tc_vecs = [(cst[pl.ds(jj * _L, _L)],
                    cst[pl.ds((_DSQ + jj) * _L, _L)]) for jj in range(_DSQ)]

        def fire(j, b):
            pltpu.async_copy(tn_hbm.at[rbuf.at[j]], ngb.at[b], sem_n[b])
            pltpu.async_copy(te_hbm.at[cbuf.at[j]], egb.at[b], sem_e[b])

        def drain(b):
            pltpu.make_async_copy(tn_hbm.at[rbuf.at[0]], ngb.at[b],
                                  sem_n[b]).wait()
            pltpu.make_async_copy(te_hbm.at[cbuf.at[0]], egb.at[b],
                                  sem_e[b]).wait()

        for b in range(_NBUF):
            fire(b, b)

        def compute(j, b):
            base = j * (_CH * _DSQ)
            nb = ngb.at[b]
            eb = egb.at[b]

            @plsc.parallel_loop(0, _CH // _L, 1, unroll=_CH // _L)
            def _grp(g):
                rows = lanes + (g * _L)

                def col(refv, k):
                    return plsc.load_gather(
                        refv, [rows, jnp.full((_L,), k, jnp.int32)])

                sx = col(nb, 4)
                qx = col(nb, 5)
                se = col(eb, 4)
                qe = col(eb, 5)
                mu = (sx + se) * (1.0 / (2.0 * _F))
                var = (qx + qe) * (1.0 / (2.0 * _F)) - mu * mu
                v = jnp.maximum(var, 0.0) + 1e-5
                iv = plsc.bitcast(v, jnp.int32)
                seed = jnp.full((_L,), 0x5F3759DF, jnp.int32)
                y = plsc.bitcast(seed - lax.shift_right_logical(iv, 1),
                                 jnp.float32)
                vh = 0.5 * v
                for _ in range(2):
                    y = y * (1.5 - vh * y * y)
                pos = base + (g * _L * _DSQ) + lanes * _DSQ
                for jj in range(_DSQ):
                    pxj = col(nb, jj)
                    pej = col(eb, jj)
                    tj, cj = tc_vecs[jj]
                    z = (pxj + pej - mu * tj) * y + cj
                    sg = 1.0 / (1.0 + jnp.exp(-z))
                    plsc.store_scatter(atb, [pos + jj], sg)

        def round_body(jj, carry):
            j0 = jj * _NBUF
            for b in range(_NBUF):
                j = j0 + b
                drain(b)
                compute(j, b)
                jn = j + _NBUF

                @pl.when(jn < nchunk)
                def _():
                    fire(jn, b)
            return carry

        lax.fori_loop(0, nchunk // _NBUF, round_body, 0)
        obase = wid * olen_t
        pltpu.sync_copy(atb, at_hbm.at[pl.ds(obase, olen_t)])

    return sck(tn, te, rowp, colp, consts)


def kernel(x, e, hyperedge_index, ln_scale, ln_bias, W, b):
    n_nodes = x.shape[0] // _D
    n_he = e.shape[0] // _D
    nnz = hyperedge_index.shape[1]
    Ws = W * ln_scale[:, None]
    t = jnp.sum(Ws, axis=0)
    cb = ln_bias @ W + b
    wx = (jnp.zeros((_F, _TBLW), jnp.float32)
          .at[:, :_DSQ].set(Ws[:_F]).at[:, 4].set(1.0))
    we = (jnp.zeros((_F, _TBLW), jnp.float32)
          .at[:, :_DSQ].set(Ws[_F:]).at[:, 4].set(1.0))
    xr = x.reshape(n_nodes, _D * _F)
    er = e.reshape(n_he, _D * _F)
    tn, te = _tc_tables(xr, er, wx, we)
    consts = jnp.repeat(jnp.concatenate([t, cb]), _L)

    grain = _NW * _CH
    nnzp = ((nnz + grain - 1) // grain) * grain
    pad = nnzp - nnz
    row = hyperedge_index[0]
    col = hyperedge_index[1]
    rowp = jnp.concatenate([row, jnp.zeros((pad,), row.dtype)])
    colp = jnp.concatenate([col, jnp.zeros((pad,), col.dtype)])
    rowp = rowp.reshape(nnzp // _CH, _CH).astype(jnp.int32)
    colp = colp.reshape(nnzp // _CH, _CH).astype(jnp.int32)

    at = _sc_sheaf(tn, te, rowp, colp, consts, nnzp)

    rf = row.astype(jnp.float32).reshape(nnz // _F, _F)
    cf = col.astype(jnp.float32).reshape(nnz // _F, _F)
    e4 = (jnp.arange(_DSQ * _F)[None, :] // _DSQ
          == jnp.arange(_F)[:, None]).astype(jnp.float32)
    i0, i1 = _tc_idx(rf, cf, e4)

    m = nnz * _DSQ
    idx = jnp.stack([i0.reshape(-1), i1.reshape(-1)])
    return idx, at[:m]


# speedup vs baseline: 12.0114x; 1.0186x over previous
import functools

import jax
import jax.numpy as jnp
from jax import lax
from jax.experimental import pallas as pl
from jax.experimental.pallas import tpu as pltpu
from jax.experimental.pallas import tpu_sc as plsc

_D = 2
_F = 128
_DSQ = _D * _D
_TBLW = 16
_NC = 2
_NS = 16
_NW = _NC * _NS
_CH = 128
_L = 16
_NBUF = 4


def _tc_tables_body(x_ref, e_ref, wx_ref, we_ref, tn_ref, te_ref):
    lane = lax.broadcasted_iota(jnp.int32, tn_ref.shape, 1)
    for src, w, dst in ((x_ref, wx_ref, tn_ref), (e_ref, we_ref, te_ref)):
        v = src[...]
        m = (v[:, :_F] + v[:, _F:]) * 0.5
        p = jnp.dot(m, w[...], preferred_element_type=jnp.float32)
        q = jnp.sum(m * m, axis=1, keepdims=True)
        dst[...] = p + jnp.where(lane == 5, q, 0.0)


def _tc_tables(xr, er, wx, we):
    n, nh = xr.shape[0], er.shape[0]
    blk = 1000
    return pl.pallas_call(
        _tc_tables_body,
        grid=(n // blk,),
        in_specs=[
            pl.BlockSpec((blk, _D * _F), lambda i: (i, 0)),
            pl.BlockSpec((blk, _D * _F), lambda i: (i, 0)),
            pl.BlockSpec((_F, _TBLW), lambda i: (0, 0)),
            pl.BlockSpec((_F, _TBLW), lambda i: (0, 0)),
        ],
        out_specs=[
            pl.BlockSpec((blk, _TBLW), lambda i: (i, 0)),
            pl.BlockSpec((blk, _TBLW), lambda i: (i, 0)),
        ],
        out_shape=[
            jax.ShapeDtypeStruct((n, _TBLW), jnp.float32),
            jax.ShapeDtypeStruct((nh, _TBLW), jnp.float32),
        ],
    )(xr, er, wx, we)


def _tc_idx_body(r_ref, c_ref, e_ref, o0_ref, o1_ref):
    lane = lax.broadcasted_iota(jnp.int32, o0_ref.shape, 1)
    k = lane & 3
    p0 = (k >> 1).astype(jnp.float32)
    p1 = (k & 1).astype(jnp.float32)
    ex = e_ref[...]
    m0 = jnp.dot(r_ref[...], ex, preferred_element_type=jnp.float32,
                 precision=lax.Precision.HIGHEST)
    m1 = jnp.dot(c_ref[...], ex, preferred_element_type=jnp.float32,
                 precision=lax.Precision.HIGHEST)
    o0_ref[...] = (2.0 * m0 + p0).astype(jnp.int32)
    o1_ref[...] = (2.0 * m1 + p1).astype(jnp.int32)


def _tc_idx(rf, cf, e4):
    n = rf.shape[0]
    return pl.pallas_call(
        _tc_idx_body,
        grid=(1,),
        in_specs=[
            pl.BlockSpec((n, _F), lambda i: (0, 0)),
            pl.BlockSpec((n, _F), lambda i: (0, 0)),
            pl.BlockSpec((_F, _DSQ * _F), lambda i: (0, 0)),
        ],
        out_specs=[
            pl.BlockSpec((n, _DSQ * _F), lambda i: (0, 0)),
            pl.BlockSpec((n, _DSQ * _F), lambda i: (0, 0)),
        ],
        out_shape=[
            jax.ShapeDtypeStruct((n, _DSQ * _F), jnp.int32),
            jax.ShapeDtypeStruct((n, _DSQ * _F), jnp.int32),
        ],
    )(rf, cf, e4)


def _tc_final_body(d0, d1, d2, d3, s_ref, q_ref, cst_ref, o_ref):
    mu = s_ref[...] * (1.0 / (2.0 * _F))
    var = q_ref[...] * (1.0 / (2.0 * _F)) - mu * mu
    inv = lax.rsqrt(jnp.maximum(var, 0.0) + 1e-5)
    rowi = lax.broadcasted_iota(jnp.int32, (_F, _DSQ * _F), 0)
    coli = lax.broadcasted_iota(jnp.int32, (_F, _DSQ * _F), 1)
    acc = None
    for j, dj in enumerate((d0, d1, d2, d3)):
        z = (dj[...] - mu * cst_ref[j]) * inv + cst_ref[_DSQ + j]
        sg = jax.nn.sigmoid(z)
        ej = (coli == _DSQ * rowi + j).astype(jnp.float32)
        part = jnp.dot(sg, ej, preferred_element_type=jnp.float32,
                       precision=lax.Precision.HIGHEST)
        acc = part if acc is None else acc + part
    o_ref[...] = acc


def _tc_final(d0, d1, d2, d3, s, q, cst8):
    n = d0.shape[0]
    vspec = pl.BlockSpec((n, _F), lambda i: (0, 0))
    return pl.pallas_call(
        _tc_final_body,
        grid=(1,),
        in_specs=[vspec] * 6 + [pl.BlockSpec(memory_space=pltpu.SMEM)],
        out_specs=pl.BlockSpec((n, _DSQ * _F), lambda i: (0, 0)),
        out_shape=jax.ShapeDtypeStruct((n, _DSQ * _F), jnp.float32),
    )(d0, d1, d2, d3, s, q, cst8)


def _sc_sheaf(tn, te, rowp, colp, nnzp):
    per_tile = nnzp // _NW
    nchunk = per_tile // _CH
    mesh = plsc.VectorSubcoreMesh(core_axis_name="c", subcore_axis_name="s")
    vec_t = jax.ShapeDtypeStruct((nnzp,), jnp.float32)

    @functools.partial(
        pl.kernel,
        mesh=mesh,
        compiler_params=pltpu.CompilerParams(needs_layout_passes=False,
                                             use_tc_tiling_on_sc=False),
        out_type=[vec_t] * 6,
        scratch_types=[
            pltpu.VMEM((nchunk, _CH), jnp.int32),
            pltpu.VMEM((nchunk, _CH), jnp.int32),
            pltpu.VMEM((_NBUF, _CH, _TBLW), jnp.float32),
            pltpu.VMEM((_NBUF, _CH, _TBLW), jnp.float32),
        ] + [pltpu.VMEM((per_tile,), jnp.float32)] * 6
          + [pltpu.SemaphoreType.DMA] * (2 * _NBUF),
    )
    def sck(tn_hbm, te_hbm, row_hbm, col_hbm,
            d0_hbm, d1_hbm, d2_hbm, d3_hbm, s_hbm, q_hbm,
            rbuf, cbuf, ngb, egb, b0, b1, b2, b3, bs, bq, *sems):
        obufs = (b0, b1, b2, b3, bs, bq)
        ohbms = (d0_hbm, d1_hbm, d2_hbm, d3_hbm, s_hbm, q_hbm)
        sem_n = sems[:_NBUF]
        sem_e = sems[_NBUF:]
        wid = lax.axis_index("s") * _NC + lax.axis_index("c")
        pltpu.sync_copy(row_hbm.at[pl.ds(wid * nchunk, nchunk)], rbuf)
        pltpu.sync_copy(col_hbm.at[pl.ds(wid * nchunk, nchunk)], cbuf)
        lanes = lax.iota(jnp.int32, _L)

        def fire(j, b):
            pltpu.async_copy(tn_hbm.at[rbuf.at[j]], ngb.at[b], sem_n[b])
            pltpu.async_copy(te_hbm.at[cbuf.at[j]], egb.at[b], sem_e[b])

        def drain(b):
            pltpu.make_async_copy(tn_hbm.at[rbuf.at[0]], ngb.at[b],
                                  sem_n[b]).wait()
            pltpu.make_async_copy(te_hbm.at[cbuf.at[0]], egb.at[b],
                                  sem_e[b]).wait()

        for b in range(_NBUF):
            fire(b, b)

        def compute(j, b):
            base = j * _CH
            nb = ngb.at[b]
            eb = egb.at[b]

            @plsc.parallel_loop(0, _CH // _L, 1, unroll=_CH // _L)
            def _grp(g):
                rows = lanes + (g * _L)
                off = base + g * _L

                def col(refv, k):
                    return plsc.load_gather(
                        refv, [rows, jnp.full((_L,), k, jnp.int32)])

                for k in range(6):
                    obufs[k][pl.ds(off, _L)] = col(nb, k) + col(eb, k)

        def round_body(jj, carry):
            j0 = jj * _NBUF
            for b in range(_NBUF):
                j = j0 + b
                drain(b)
                compute(j, b)
                jn = j + _NBUF

                @pl.when(jn < nchunk)
                def _():
                    fire(jn, b)
            return carry

        lax.fori_loop(0, nchunk // _NBUF, round_body, 0)
        obase = wid * per_tile
        for k in range(6):
            pltpu.sync_copy(obufs[k], ohbms[k].at[pl.ds(obase, per_tile)])

    return sck(tn, te, rowp, colp)


def kernel(x, e, hyperedge_index, ln_scale, ln_bias, W, b):
    n_nodes = x.shape[0] // _D
    n_he = e.shape[0] // _D
    nnz = hyperedge_index.shape[1]
    Ws = W * ln_scale[:, None]
    t = jnp.sum(Ws, axis=0)
    cb = ln_bias @ W + b
    wx = (jnp.zeros((_F, _TBLW), jnp.float32)
          .at[:, :_DSQ].set(Ws[:_F]).at[:, 4].set(1.0))
    we = (jnp.zeros((_F, _TBLW), jnp.float32)
          .at[:, :_DSQ].set(Ws[_F:]).at[:, 4].set(1.0))
    xr = x.reshape(n_nodes, _D * _F)
    er = e.reshape(n_he, _D * _F)
    tn, te = _tc_tables(xr, er, wx, we)

    grain = _NW * _CH
    nnzp = ((nnz + grain - 1) // grain) * grain
    pad = nnzp - nnz
    row = hyperedge_index[0]
    col = hyperedge_index[1]
    rowp = jnp.concatenate([row, jnp.zeros((pad,), row.dtype)])
    colp = jnp.concatenate([col, jnp.zeros((pad,), col.dtype)])
    rowp = rowp.reshape(nnzp // _CH, _CH).astype(jnp.int32)
    colp = colp.reshape(nnzp // _CH, _CH).astype(jnp.int32)

    d0, d1, d2, d3, s, q = _sc_sheaf(tn, te, rowp, colp, nnzp)
    cst8 = jnp.concatenate([t, cb])
    nr = nnzp // _F
    at2 = _tc_final(d0.reshape(nr, _F), d1.reshape(nr, _F),
                    d2.reshape(nr, _F), d3.reshape(nr, _F),
                    s.reshape(nr, _F), q.reshape(nr, _F), cst8)
    at = at2.reshape(-1)

    rf = row.astype(jnp.float32).reshape(nnz // _F, _F)
    cf = col.astype(jnp.float32).reshape(nnz // _F, _F)
    e4 = (jnp.arange(_DSQ * _F)[None, :] // _DSQ
          == jnp.arange(_F)[:, None]).astype(jnp.float32)
    i0, i1 = _tc_idx(rf, cf, e4)

    m = nnz * _DSQ
    idx = jnp.stack([i0.reshape(-1), i1.reshape(-1)])
    return idx, at[:m]


# speedup vs baseline: 13.7068x; 1.1412x over previous
import functools

import jax
import jax.numpy as jnp
from jax import lax
from jax.experimental import pallas as pl
from jax.experimental.pallas import tpu as pltpu
from jax.experimental.pallas import tpu_sc as plsc

_D = 2
_F = 128
_DSQ = _D * _D
_TBLW = 16
_NC = 2
_NS = 16
_NW = _NC * _NS
_CH = 128
_L = 16
_NBUF = 4


def _tc_tables_body(x_ref, e_ref, wx_ref, we_ref, tn_ref, te_ref):
    lane = lax.broadcasted_iota(jnp.int32, tn_ref.shape, 1)
    for src, w, dst in ((x_ref, wx_ref, tn_ref), (e_ref, we_ref, te_ref)):
        v = src[...]
        blk = v.shape[0] // _D
        v3 = v.reshape(blk, _D, _F)
        m = (v3[:, 0, :] + v3[:, 1, :]) * 0.5
        p = jnp.dot(m, w[...], preferred_element_type=jnp.float32)
        q = jnp.sum(m * m, axis=1, keepdims=True)
        dst[...] = p + jnp.where(lane == 5, q, 0.0)


def _tc_tables(x, e, wx, we):
    n, nh = x.shape[0] // _D, e.shape[0] // _D
    blk = 2000
    return pl.pallas_call(
        _tc_tables_body,
        grid=(n // blk,),
        in_specs=[
            pl.BlockSpec((_D * blk, _F), lambda i: (i, 0)),
            pl.BlockSpec((_D * blk, _F), lambda i: (i, 0)),
            pl.BlockSpec((_F, _TBLW), lambda i: (0, 0)),
            pl.BlockSpec((_F, _TBLW), lambda i: (0, 0)),
        ],
        out_specs=[
            pl.BlockSpec((blk, _TBLW), lambda i: (i, 0)),
            pl.BlockSpec((blk, _TBLW), lambda i: (i, 0)),
        ],
        out_shape=[
            jax.ShapeDtypeStruct((n, _TBLW), jnp.float32),
            jax.ShapeDtypeStruct((nh, _TBLW), jnp.float32),
        ],
    )(x, e, wx, we)


def _tc_idx_body(r_ref, c_ref, e_ref, o0_ref, o1_ref):
    lane = lax.broadcasted_iota(jnp.int32, o0_ref.shape, 1)
    k = lane & 3
    p0 = (k >> 1).astype(jnp.float32)
    p1 = (k & 1).astype(jnp.float32)
    ex = e_ref[...]
    m0 = jnp.dot(r_ref[...], ex, preferred_element_type=jnp.float32,
                 precision=lax.Precision.HIGHEST)
    m1 = jnp.dot(c_ref[...], ex, preferred_element_type=jnp.float32,
                 precision=lax.Precision.HIGHEST)
    o0_ref[...] = (2.0 * m0 + p0).astype(jnp.int32)
    o1_ref[...] = (2.0 * m1 + p1).astype(jnp.int32)


def _tc_idx(rf, cf, e4):
    n = rf.shape[0]
    return pl.pallas_call(
        _tc_idx_body,
        grid=(1,),
        in_specs=[
            pl.BlockSpec((n, _F), lambda i: (0, 0)),
            pl.BlockSpec((n, _F), lambda i: (0, 0)),
            pl.BlockSpec((_F, _DSQ * _F), lambda i: (0, 0)),
        ],
        out_specs=[
            pl.BlockSpec((n, _DSQ * _F), lambda i: (0, 0)),
            pl.BlockSpec((n, _DSQ * _F), lambda i: (0, 0)),
        ],
        out_shape=[
            jax.ShapeDtypeStruct((n, _DSQ * _F), jnp.int32),
            jax.ShapeDtypeStruct((n, _DSQ * _F), jnp.int32),
        ],
    )(rf, cf, e4)


def _tc_final_body(d0, d1, d2, d3, s_ref, q_ref, cst_ref, o_ref):
    mu = s_ref[...] * (1.0 / (2.0 * _F))
    var = q_ref[...] * (1.0 / (2.0 * _F)) - mu * mu
    inv = lax.rsqrt(jnp.maximum(var, 0.0) + 1e-5)
    rowi = lax.broadcasted_iota(jnp.int32, (_F, _DSQ * _F), 0)
    coli = lax.broadcasted_iota(jnp.int32, (_F, _DSQ * _F), 1)
    acc = None
    for j, dj in enumerate((d0, d1, d2, d3)):
        z = (dj[...] - mu * cst_ref[j]) * inv + cst_ref[_DSQ + j]
        sg = jax.nn.sigmoid(z)
        ej = (coli == _DSQ * rowi + j).astype(jnp.float32)
        part = jnp.dot(sg, ej, preferred_element_type=jnp.float32)
        acc = part if acc is None else acc + part
    o_ref[...] = acc


def _tc_final(d0, d1, d2, d3, s, q, cst8):
    n = d0.shape[0]
    vspec = pl.BlockSpec((n, _F), lambda i: (0, 0))
    return pl.pallas_call(
        _tc_final_body,
        grid=(1,),
        in_specs=[vspec] * 6 + [pl.BlockSpec(memory_space=pltpu.SMEM)],
        out_specs=pl.BlockSpec((n, _DSQ * _F), lambda i: (0, 0)),
        out_shape=jax.ShapeDtypeStruct((n, _DSQ * _F), jnp.float32),
    )(d0, d1, d2, d3, s, q, cst8)


def _sc_sheaf(tn, te, rowp, colp, nnzp):
    per_tile = nnzp // _NW
    nchunk = per_tile // _CH
    mesh = plsc.VectorSubcoreMesh(core_axis_name="c", subcore_axis_name="s")
    vec_t = jax.ShapeDtypeStruct((nnzp,), jnp.float32)

    @functools.partial(
        pl.kernel,
        mesh=mesh,
        compiler_params=pltpu.CompilerParams(needs_layout_passes=False,
                                             use_tc_tiling_on_sc=False),
        out_type=[vec_t] * 6,
        scratch_types=[
            pltpu.VMEM((nchunk, _CH), jnp.int32),
            pltpu.VMEM((nchunk, _CH), jnp.int32),
            pltpu.VMEM((_NBUF, _CH, _TBLW), jnp.float32),
            pltpu.VMEM((_NBUF, _CH, _TBLW), jnp.float32),
        ] + [pltpu.VMEM((per_tile,), jnp.float32)] * 6
          + [pltpu.SemaphoreType.DMA] * (2 * _NBUF),
    )
    def sck(tn_hbm, te_hbm, row_hbm, col_hbm,
            d0_hbm, d1_hbm, d2_hbm, d3_hbm, s_hbm, q_hbm,
            rbuf, cbuf, ngb, egb, b0, b1, b2, b3, bs, bq, *sems):
        obufs = (b0, b1, b2, b3, bs, bq)
        ohbms = (d0_hbm, d1_hbm, d2_hbm, d3_hbm, s_hbm, q_hbm)
        sem_n = sems[:_NBUF]
        sem_e = sems[_NBUF:]
        wid = lax.axis_index("s") * _NC + lax.axis_index("c")
        pltpu.sync_copy(row_hbm.at[pl.ds(wid * nchunk, nchunk)], rbuf)
        pltpu.sync_copy(col_hbm.at[pl.ds(wid * nchunk, nchunk)], cbuf)
        lanes = lax.iota(jnp.int32, _L)

        def fire(j, b):
            pltpu.async_copy(tn_hbm.at[rbuf.at[j]], ngb.at[b], sem_n[b])
            pltpu.async_copy(te_hbm.at[cbuf.at[j]], egb.at[b], sem_e[b])

        def drain(b):
            pltpu.make_async_copy(tn_hbm.at[rbuf.at[0]], ngb.at[b],
                                  sem_n[b]).wait()
            pltpu.make_async_copy(te_hbm.at[cbuf.at[0]], egb.at[b],
                                  sem_e[b]).wait()

        for b in range(_NBUF):
            fire(b, b)

        def compute(j, b):
            base = j * _CH
            nb = ngb.at[b]
            eb = egb.at[b]

            @plsc.parallel_loop(0, _CH // _L, 1, unroll=_CH // _L)
            def _grp(g):
                rows = lanes + (g * _L)
                off = base + g * _L

                def col(refv, k):
                    return plsc.load_gather(
                        refv, [rows, jnp.full((_L,), k, jnp.int32)])

                for k in range(6):
                    obufs[k][pl.ds(off, _L)] = col(nb, k) + col(eb, k)

        def round_body(jj, carry):
            j0 = jj * _NBUF
            for b in range(_NBUF):
                j = j0 + b
                drain(b)
                compute(j, b)
                jn = j + _NBUF

                @pl.when(jn < nchunk)
                def _():
                    fire(jn, b)
            return carry

        lax.fori_loop(0, nchunk // _NBUF, round_body, 0)
        obase = wid * per_tile
        for k in range(6):
            pltpu.sync_copy(obufs[k], ohbms[k].at[pl.ds(obase, per_tile)])

    return sck(tn, te, rowp, colp)


def kernel(x, e, hyperedge_index, ln_scale, ln_bias, W, b):
    n_nodes = x.shape[0] // _D
    n_he = e.shape[0] // _D
    nnz = hyperedge_index.shape[1]
    Ws = W * ln_scale[:, None]
    t = jnp.sum(Ws, axis=0)
    cb = ln_bias @ W + b
    wx = (jnp.zeros((_F, _TBLW), jnp.float32)
          .at[:, :_DSQ].set(Ws[:_F]).at[:, 4].set(1.0))
    we = (jnp.zeros((_F, _TBLW), jnp.float32)
          .at[:, :_DSQ].set(Ws[_F:]).at[:, 4].set(1.0))
    tn, te = _tc_tables(x, e, wx, we)

    grain = _NW * _CH
    nnzp = ((nnz + grain - 1) // grain) * grain
    pad = nnzp - nnz
    row = hyperedge_index[0]
    col = hyperedge_index[1]
    rowp = jnp.concatenate([row, jnp.zeros((pad,), row.dtype)])
    colp = jnp.concatenate([col, jnp.zeros((pad,), col.dtype)])
    rowp = rowp.reshape(nnzp // _CH, _CH).astype(jnp.int32)
    colp = colp.reshape(nnzp // _CH, _CH).astype(jnp.int32)

    d0, d1, d2, d3, s, q = _sc_sheaf(tn, te, rowp, colp, nnzp)
    cst8 = jnp.concatenate([t, cb])
    nr = nnzp // _F
    at2 = _tc_final(d0.reshape(nr, _F), d1.reshape(nr, _F),
                    d2.reshape(nr, _F), d3.reshape(nr, _F),
                    s.reshape(nr, _F), q.reshape(nr, _F), cst8)
    at = at2.reshape(-1)

    rf = row.astype(jnp.float32).reshape(nnz // _F, _F)
    cf = col.astype(jnp.float32).reshape(nnz // _F, _F)
    e4 = (jnp.arange(_DSQ * _F)[None, :] // _DSQ
          == jnp.arange(_F)[:, None]).astype(jnp.float32)
    i0, i1 = _tc_idx(rf, cf, e4)

    m = nnz * _DSQ
    idx = jnp.stack([i0.reshape(-1), i1.reshape(-1)])
    return idx, at[:m]


# speedup vs baseline: 14.5923x; 1.0646x over previous
import functools

import jax
import jax.numpy as jnp
from jax import lax
from jax.experimental import pallas as pl
from jax.experimental.pallas import tpu as pltpu
from jax.experimental.pallas import tpu_sc as plsc

_D = 2
_F = 128
_DSQ = _D * _D
_TBLW = 16
_NC = 2
_NS = 16
_NW = _NC * _NS
_CH = 128
_L = 16
_NBUF = 4


def _tc_tables_body(x_ref, e_ref, wx_ref, we_ref, tn_ref, te_ref):
    lane = lax.broadcasted_iota(jnp.int32, tn_ref.shape, 1)
    for src, w, dst in ((x_ref, wx_ref, tn_ref), (e_ref, we_ref, te_ref)):
        v = src[...]
        m = (v + jnp.roll(v, -1, axis=0)) * 0.5
        p = jnp.dot(m, w[...], preferred_element_type=jnp.float32)
        q = jnp.sum(m * m, axis=1, keepdims=True)
        dst[...] = p + jnp.where(lane == 5, q, 0.0)


def _tc_tables(x, e, wx, we):
    n2, nh2 = x.shape[0], e.shape[0]
    blk = 4000
    return pl.pallas_call(
        _tc_tables_body,
        grid=(n2 // blk,),
        in_specs=[
            pl.BlockSpec((blk, _F), lambda i: (i, 0)),
            pl.BlockSpec((blk, _F), lambda i: (i, 0)),
            pl.BlockSpec((_F, _TBLW), lambda i: (0, 0)),
            pl.BlockSpec((_F, _TBLW), lambda i: (0, 0)),
        ],
        out_specs=[
            pl.BlockSpec((blk, _TBLW), lambda i: (i, 0)),
            pl.BlockSpec((blk, _TBLW), lambda i: (i, 0)),
        ],
        out_shape=[
            jax.ShapeDtypeStruct((n2, _TBLW), jnp.float32),
            jax.ShapeDtypeStruct((nh2, _TBLW), jnp.float32),
        ],
    )(x, e, wx, we)


def _tc_idx_body(r_ref, c_ref, e_ref, o0_ref, o1_ref):
    lane = lax.broadcasted_iota(jnp.int32, o0_ref.shape, 1)
    k = lane & 3
    p0 = (k >> 1).astype(jnp.float32)
    p1 = (k & 1).astype(jnp.float32)
    ex = e_ref[...]
    m0 = jnp.dot(r_ref[...], ex, preferred_element_type=jnp.float32,
                 precision=lax.Precision.HIGHEST)
    m1 = jnp.dot(c_ref[...], ex, preferred_element_type=jnp.float32,
                 precision=lax.Precision.HIGHEST)
    o0_ref[...] = (2.0 * m0 + p0).astype(jnp.int32)
    o1_ref[...] = (2.0 * m1 + p1).astype(jnp.int32)


def _tc_idx(rf, cf, e4):
    n = rf.shape[0]
    return pl.pallas_call(
        _tc_idx_body,
        grid=(1,),
        in_specs=[
            pl.BlockSpec((n, _F), lambda i: (0, 0)),
            pl.BlockSpec((n, _F), lambda i: (0, 0)),
            pl.BlockSpec((_F, _DSQ * _F), lambda i: (0, 0)),
        ],
        out_specs=[
            pl.BlockSpec((n, _DSQ * _F), lambda i: (0, 0)),
            pl.BlockSpec((n, _DSQ * _F), lambda i: (0, 0)),
        ],
        out_shape=[
            jax.ShapeDtypeStruct((n, _DSQ * _F), jnp.int32),
            jax.ShapeDtypeStruct((n, _DSQ * _F), jnp.int32),
        ],
    )(rf, cf, e4)


def _tc_final_body(d0, d1, d2, d3, s_ref, q_ref, cst_ref, o_ref):
    mu = s_ref[...] * (1.0 / (2.0 * _F))
    var = q_ref[...] * (1.0 / (2.0 * _F)) - mu * mu
    inv = lax.rsqrt(jnp.maximum(var, 0.0) + 1e-5)
    rowi = lax.broadcasted_iota(jnp.int32, (_F, _DSQ * _F), 0)
    coli = lax.broadcasted_iota(jnp.int32, (_F, _DSQ * _F), 1)
    acc = None
    for j, dj in enumerate((d0, d1, d2, d3)):
        z = (dj[...] - mu * cst_ref[j]) * inv + cst_ref[_DSQ + j]
        sg = jax.nn.sigmoid(z)
        ej = (coli == _DSQ * rowi + j).astype(jnp.float32)
        part = jnp.dot(sg, ej, preferred_element_type=jnp.float32)
        acc = part if acc is None else acc + part
    o_ref[...] = acc


def _tc_final(d0, d1, d2, d3, s, q, cst8):
    n = d0.shape[0]
    vspec = pl.BlockSpec((n, _F), lambda i: (0, 0))
    return pl.pallas_call(
        _tc_final_body,
        grid=(1,),
        in_specs=[vspec] * 6 + [pl.BlockSpec(memory_space=pltpu.SMEM)],
        out_specs=pl.BlockSpec((n, _DSQ * _F), lambda i: (0, 0)),
        out_shape=jax.ShapeDtypeStruct((n, _DSQ * _F), jnp.float32),
    )(d0, d1, d2, d3, s, q, cst8)


def _sc_sheaf(tn, te, rowp, colp, nnzp):
    per_tile = nnzp // _NW
    nchunk = per_tile // _CH
    mesh = plsc.VectorSubcoreMesh(core_axis_name="c", subcore_axis_name="s")
    vec_t = jax.ShapeDtypeStruct((nnzp,), jnp.float32)

    @functools.partial(
        pl.kernel,
        mesh=mesh,
        compiler_params=pltpu.CompilerParams(needs_layout_passes=False,
                                             use_tc_tiling_on_sc=False),
        out_type=[vec_t] * 6,
        scratch_types=[
            pltpu.VMEM((per_tile,), jnp.int32),
            pltpu.VMEM((per_tile,), jnp.int32),
            pltpu.VMEM((_NBUF, _CH, _TBLW), jnp.float32),
            pltpu.VMEM((_NBUF, _CH, _TBLW), jnp.float32),
        ] + [pltpu.VMEM((per_tile,), jnp.float32)] * 6
          + [pltpu.SemaphoreType.DMA] * (2 * _NBUF),
    )
    def sck(tn_hbm, te_hbm, row_hbm, col_hbm,
            d0_hbm, d1_hbm, d2_hbm, d3_hbm, s_hbm, q_hbm,
            rbuf, cbuf, ngb, egb, b0, b1, b2, b3, bs, bq, *sems):
        obufs = (b0, b1, b2, b3, bs, bq)
        ohbms = (d0_hbm, d1_hbm, d2_hbm, d3_hbm, s_hbm, q_hbm)
        sem_n = sems[:_NBUF]
        sem_e = sems[_NBUF:]
        wid = lax.axis_index("s") * _NC + lax.axis_index("c")
        pltpu.sync_copy(row_hbm.at[pl.ds(wid * per_tile, per_tile)], rbuf)
        pltpu.sync_copy(col_hbm.at[pl.ds(wid * per_tile, per_tile)], cbuf)
        lanes = lax.iota(jnp.int32, _L)

        def fire(j, b):
            pltpu.async_copy(tn_hbm.at[rbuf.at[pl.ds(j * _CH, _CH)]],
                             ngb.at[b], sem_n[b])
            pltpu.async_copy(te_hbm.at[cbuf.at[pl.ds(j * _CH, _CH)]],
                             egb.at[b], sem_e[b])

        def drain(b):
            pltpu.make_async_copy(tn_hbm.at[rbuf.at[pl.ds(0, _CH)]],
                                  ngb.at[b], sem_n[b]).wait()
            pltpu.make_async_copy(te_hbm.at[cbuf.at[pl.ds(0, _CH)]],
                                  egb.at[b], sem_e[b]).wait()

        for b in range(_NBUF):
            fire(b, b)

        def compute(j, b):
            base = j * _CH
            nb = ngb.at[b]
            eb = egb.at[b]

            @plsc.parallel_loop(0, _CH // _L, 1, unroll=_CH // _L)
            def _grp(g):
                rows = lanes + (g * _L)
                off = base + g * _L

                def col(refv, k):
                    return plsc.load_gather(
                        refv, [rows, jnp.full((_L,), k, jnp.int32)])

                for k in range(6):
                    obufs[k][pl.ds(off, _L)] = col(nb, k) + col(eb, k)

        def round_body(jj, carry):
            j0 = jj * _NBUF
            for b in range(_NBUF):
                j = j0 + b
                drain(b)
                compute(j, b)
                jn = j + _NBUF

                @pl.when(jn < nchunk)
                def _():
                    fire(jn, b)
            return carry

        lax.fori_loop(0, nchunk // _NBUF, round_body, 0)
        obase = wid * per_tile
        for k in range(6):
            pltpu.sync_copy(obufs[k], ohbms[k].at[pl.ds(obase, per_tile)])

    return sck(tn, te, rowp, colp)


def kernel(x, e, hyperedge_index, ln_scale, ln_bias, W, b):
    n_nodes = x.shape[0] // _D
    n_he = e.shape[0] // _D
    nnz = hyperedge_index.shape[1]
    Ws = W * ln_scale[:, None]
    t = jnp.sum(Ws, axis=0)
    cb = ln_bias @ W + b
    wx = (jnp.zeros((_F, _TBLW), jnp.float32)
          .at[:, :_DSQ].set(Ws[:_F]).at[:, 4].set(1.0))
    we = (jnp.zeros((_F, _TBLW), jnp.float32)
          .at[:, :_DSQ].set(Ws[_F:]).at[:, 4].set(1.0))
    tn, te = _tc_tables(x, e, wx, we)

    grain = _NW * _CH
    nnzp = ((nnz + grain - 1) // grain) * grain
    pad = nnzp - nnz
    row = hyperedge_index[0]
    col = hyperedge_index[1]
    rowp = jnp.concatenate([row * 2, jnp.zeros((pad,), row.dtype)])
    colp = jnp.concatenate([col * 2, jnp.zeros((pad,), col.dtype)])
    rowp = rowp.astype(jnp.int32)
    colp = colp.astype(jnp.int32)

    d0, d1, d2, d3, s, q = _sc_sheaf(tn, te, rowp, colp, nnzp)
    cst8 = jnp.concatenate([t, cb])
    nr = nnzp // _F
    at2 = _tc_final(d0.reshape(nr, _F), d1.reshape(nr, _F),
                    d2.reshape(nr, _F), d3.reshape(nr, _F),
                    s.reshape(nr, _F), q.reshape(nr, _F), cst8)
    at = at2.reshape(-1)

    rf = row.astype(jnp.float32).reshape(nnz // _F, _F)
    cf = col.astype(jnp.float32).reshape(nnz // _F, _F)
    e4 = (jnp.arange(_DSQ * _F)[None, :] // _DSQ
          == jnp.arange(_F)[:, None]).astype(jnp.float32)
    i0, i1 = _tc_idx(rf, cf, e4)

    m = nnz * _DSQ
    idx = jnp.stack([i0.reshape(-1), i1.reshape(-1)])
    return idx, at[:m]


# speedup vs baseline: 14.5948x; 1.0002x over previous
import functools

import jax
import jax.numpy as jnp
from jax import lax
from jax.experimental import pallas as pl
from jax.experimental.pallas import tpu as pltpu
from jax.experimental.pallas import tpu_sc as plsc

_D = 2
_F = 128
_DSQ = _D * _D
_TBLW = 16
_NC = 2
_NS = 16
_NW = _NC * _NS
_CH = 128
_L = 16
_NBUF = 8


def _tc_tables_body(x_ref, e_ref, wx_ref, we_ref, tn_ref, te_ref):
    lane = lax.broadcasted_iota(jnp.int32, tn_ref.shape, 1)
    for src, w, dst in ((x_ref, wx_ref, tn_ref), (e_ref, we_ref, te_ref)):
        v = src[...]
        m = (v + jnp.roll(v, -1, axis=0)) * 0.5
        p = jnp.dot(m, w[...], preferred_element_type=jnp.float32)
        q = jnp.sum(m * m, axis=1, keepdims=True)
        dst[...] = p + jnp.where(lane == 5, q, 0.0)


def _tc_tables(x, e, wx, we):
    n2, nh2 = x.shape[0], e.shape[0]
    blk = 4000
    return pl.pallas_call(
        _tc_tables_body,
        grid=(n2 // blk,),
        in_specs=[
            pl.BlockSpec((blk, _F), lambda i: (i, 0)),
            pl.BlockSpec((blk, _F), lambda i: (i, 0)),
            pl.BlockSpec((_F, _TBLW), lambda i: (0, 0)),
            pl.BlockSpec((_F, _TBLW), lambda i: (0, 0)),
        ],
        out_specs=[
            pl.BlockSpec((blk, _TBLW), lambda i: (i, 0)),
            pl.BlockSpec((blk, _TBLW), lambda i: (i, 0)),
        ],
        out_shape=[
            jax.ShapeDtypeStruct((n2, _TBLW), jnp.float32),
            jax.ShapeDtypeStruct((nh2, _TBLW), jnp.float32),
        ],
    )(x, e, wx, we)


def _tc_idx_body(r_ref, c_ref, e_ref, o0_ref, o1_ref):
    lane = lax.broadcasted_iota(jnp.int32, o0_ref.shape, 1)
    k = lane & 3
    p0 = (k >> 1).astype(jnp.float32)
    p1 = (k & 1).astype(jnp.float32)
    ex = e_ref[...]
    m0 = jnp.dot(r_ref[...], ex, preferred_element_type=jnp.float32,
                 precision=lax.Precision.HIGHEST)
    m1 = jnp.dot(c_ref[...], ex, preferred_element_type=jnp.float32,
                 precision=lax.Precision.HIGHEST)
    o0_ref[...] = (2.0 * m0 + p0).astype(jnp.int32)
    o1_ref[...] = (2.0 * m1 + p1).astype(jnp.int32)


def _tc_idx(rf, cf, e4):
    n = rf.shape[0]
    return pl.pallas_call(
        _tc_idx_body,
        grid=(1,),
        in_specs=[
            pl.BlockSpec((n, _F), lambda i: (0, 0)),
            pl.BlockSpec((n, _F), lambda i: (0, 0)),
            pl.BlockSpec((_F, _DSQ * _F), lambda i: (0, 0)),
        ],
        out_specs=[
            pl.BlockSpec((n, _DSQ * _F), lambda i: (0, 0)),
            pl.BlockSpec((n, _DSQ * _F), lambda i: (0, 0)),
        ],
        out_shape=[
            jax.ShapeDtypeStruct((n, _DSQ * _F), jnp.int32),
            jax.ShapeDtypeStruct((n, _DSQ * _F), jnp.int32),
        ],
    )(rf, cf, e4)


def _tc_final_body(d0, d1, d2, d3, s_ref, q_ref, cst_ref, o_ref):
    mu = s_ref[...] * (1.0 / (2.0 * _F))
    var = q_ref[...] * (1.0 / (2.0 * _F)) - mu * mu
    inv = lax.rsqrt(jnp.maximum(var, 0.0) + 1e-5)
    rowi = lax.broadcasted_iota(jnp.int32, (_F, _DSQ * _F), 0)
    coli = lax.broadcasted_iota(jnp.int32, (_F, _DSQ * _F), 1)
    acc = None
    for j, dj in enumerate((d0, d1, d2, d3)):
        z = (dj[...] - mu * cst_ref[j]) * inv + cst_ref[_DSQ + j]
        sg = jax.nn.sigmoid(z)
        ej = (coli == _DSQ * rowi + j).astype(jnp.float32)
        part = jnp.dot(sg, ej, preferred_element_type=jnp.float32)
        acc = part if acc is None else acc + part
    o_ref[...] = acc


def _tc_final(d0, d1, d2, d3, s, q, cst8):
    n = d0.shape[0]
    vspec = pl.BlockSpec((n, _F), lambda i: (0, 0))
    return pl.pallas_call(
        _tc_final_body,
        grid=(1,),
        in_specs=[vspec] * 6 + [pl.BlockSpec(memory_space=pltpu.SMEM)],
        out_specs=pl.BlockSpec((n, _DSQ * _F), lambda i: (0, 0)),
        out_shape=jax.ShapeDtypeStruct((n, _DSQ * _F), jnp.float32),
    )(d0, d1, d2, d3, s, q, cst8)


def _sc_sheaf(tn, te, rowp, colp, nnzp):
    per_tile = nnzp // _NW
    nchunk = per_tile // _CH
    mesh = plsc.VectorSubcoreMesh(core_axis_name="c", subcore_axis_name="s")
    vec_t = jax.ShapeDtypeStruct((nnzp,), jnp.float32)

    @functools.partial(
        pl.kernel,
        mesh=mesh,
        compiler_params=pltpu.CompilerParams(needs_layout_passes=False,
                                             use_tc_tiling_on_sc=False),
        out_type=[vec_t] * 6,
        scratch_types=[
            pltpu.VMEM((per_tile,), jnp.int32),
            pltpu.VMEM((per_tile,), jnp.int32),
            pltpu.VMEM((_NBUF, _CH, _TBLW), jnp.float32),
            pltpu.VMEM((_NBUF, _CH, _TBLW), jnp.float32),
        ] + [pltpu.VMEM((per_tile,), jnp.float32)] * 6
          + [pltpu.SemaphoreType.DMA] * (2 * _NBUF),
    )
    def sck(tn_hbm, te_hbm, row_hbm, col_hbm,
            d0_hbm, d1_hbm, d2_hbm, d3_hbm, s_hbm, q_hbm,
            rbuf, cbuf, ngb, egb, b0, b1, b2, b3, bs, bq, *sems):
        obufs = (b0, b1, b2, b3, bs, bq)
        ohbms = (d0_hbm, d1_hbm, d2_hbm, d3_hbm, s_hbm, q_hbm)
        sem_n = sems[:_NBUF]
        sem_e = sems[_NBUF:]
        wid = lax.axis_index("s") * _NC + lax.axis_index("c")
        pltpu.sync_copy(row_hbm.at[pl.ds(wid * per_tile, per_tile)], rbuf)
        pltpu.sync_copy(col_hbm.at[pl.ds(wid * per_tile, per_tile)], cbuf)
        lanes = lax.iota(jnp.int32, _L)

        def fire(j, b):
            pltpu.async_copy(tn_hbm.at[rbuf.at[pl.ds(j * _CH, _CH)]],
                             ngb.at[b], sem_n[b])
            pltpu.async_copy(te_hbm.at[cbuf.at[pl.ds(j * _CH, _CH)]],
                             egb.at[b], sem_e[b])

        def drain(b):
            pltpu.make_async_copy(tn_hbm.at[rbuf.at[pl.ds(0, _CH)]],
                                  ngb.at[b], sem_n[b]).wait()
            pltpu.make_async_copy(te_hbm.at[cbuf.at[pl.ds(0, _CH)]],
                                  egb.at[b], sem_e[b]).wait()

        for b in range(_NBUF):
            fire(b, b)

        def compute(j, b):
            base = j * _CH
            nb = ngb.at[b]
            eb = egb.at[b]

            @plsc.parallel_loop(0, _CH // _L, 1, unroll=_CH // _L)
            def _grp(g):
                rows = lanes + (g * _L)
                off = base + g * _L

                def col(refv, k):
                    return plsc.load_gather(
                        refv, [rows, jnp.full((_L,), k, jnp.int32)])

                for k in range(6):
                    obufs[k][pl.ds(off, _L)] = col(nb, k) + col(eb, k)

        def round_body(jj, carry):
            j0 = jj * _NBUF
            for b in range(_NBUF):
                j = j0 + b
                drain(b)
                compute(j, b)
                jn = j + _NBUF

                @pl.when(jn < nchunk)
                def _():
                    fire(jn, b)
            return carry

        lax.fori_loop(0, nchunk // _NBUF, round_body, 0)
        obase = wid * per_tile
        for k in range(6):
            pltpu.sync_copy(obufs[k], ohbms[k].at[pl.ds(obase, per_tile)])

    return sck(tn, te, rowp, colp)


def kernel(x, e, hyperedge_index, ln_scale, ln_bias, W, b):
    n_nodes = x.shape[0] // _D
    n_he = e.shape[0] // _D
    nnz = hyperedge_index.shape[1]
    Ws = W * ln_scale[:, None]
    t = jnp.sum(Ws, axis=0)
    cb = ln_bias @ W + b
    wx = (jnp.zeros((_F, _TBLW), jnp.float32)
          .at[:, :_DSQ].set(Ws[:_F]).at[:, 4].set(1.0))
    we = (jnp.zeros((_F, _TBLW), jnp.float32)
          .at[:, :_DSQ].set(Ws[_F:]).at[:, 4].set(1.0))
    tn, te = _tc_tables(x, e, wx, we)

    grain = _NW * _CH
    nnzp = ((nnz + grain - 1) // grain) * grain
    pad = nnzp - nnz
    row = hyperedge_index[0]
    col = hyperedge_index[1]
    rowp = jnp.concatenate([row * 2, jnp.zeros((pad,), row.dtype)])
    colp = jnp.concatenate([col * 2, jnp.zeros((pad,), col.dtype)])
    rowp = rowp.astype(jnp.int32)
    colp = colp.astype(jnp.int32)

    d0, d1, d2, d3, s, q = _sc_sheaf(tn, te, rowp, colp, nnzp)
    cst8 = jnp.concatenate([t, cb])
    nr = nnzp // _F
    at2 = _tc_final(d0.reshape(nr, _F), d1.reshape(nr, _F),
                    d2.reshape(nr, _F), d3.reshape(nr, _F),
                    s.reshape(nr, _F), q.reshape(nr, _F), cst8)
    at = at2.reshape(-1)

    rf = row.astype(jnp.float32).reshape(nnz // _F, _F)
    cf = col.astype(jnp.float32).reshape(nnz // _F, _F)
    e4 = (jnp.arange(_DSQ * _F)[None, :] // _DSQ
          == jnp.arange(_F)[:, None]).astype(jnp.float32)
    i0, i1 = _tc_idx(rf, cf, e4)

    m = nnz * _DSQ
    idx = jnp.stack([i0.reshape(-1), i1.reshape(-1)])
    return idx, at[:m]
